# baseline (speedup 1.0000x reference)
import functools

import jax
import jax.numpy as jnp
from jax import lax
from jax.experimental import pallas as pl
from jax.experimental.pallas import tpu as pltpu
from jax.experimental.pallas import tpu_sc as plsc


def _sc_gather(table, idx):
    B = idx.shape[0]
    V, D = table.shape
    info = plsc.get_sparse_core_info()
    NC, NS = info.num_cores, info.num_subcores
    NW = NC * NS
    b_per_w = B // NW
    mesh = plsc.VectorSubcoreMesh(core_axis_name="c", subcore_axis_name="s")

    @functools.partial(
        pl.kernel,
        mesh=mesh,
        compiler_params=pltpu.CompilerParams(use_tc_tiling_on_sc=False),
        out_type=jax.ShapeDtypeStruct((B, D), jnp.float32),
        scratch_types=[
            pltpu.VMEM((b_per_w,), jnp.int32),
            pltpu.VMEM((b_per_w, D), jnp.float32),
            pltpu.SemaphoreType.DMA,
        ],
    )
    def gather_kernel(idx_hbm, table_hbm, out_hbm, idx_v, rows_v, sem):
        wid = lax.axis_index("s") * NC + lax.axis_index("c")
        base = wid * b_per_w
        pltpu.sync_copy(idx_hbm.at[pl.ds(base, b_per_w)], idx_v)
        pltpu.async_copy(table_hbm.at[idx_v], rows_v, sem).wait()
        pltpu.sync_copy(rows_v, out_hbm.at[pl.ds(base, b_per_w)])

    return gather_kernel(idx, table)


_BT = 2048


def _mlp_body(u_ref, d_ref, h_ref, m_ref, dtab_ref, htab_ref, w1u_ref,
              w1d_ref, w1h_ref, w1m_ref, b1_ref, w2_ref, b2_ref, w3_ref,
              b3_ref, o_ref):
    f32 = jnp.float32
    bt = u_ref.shape[0]
    ncat = dtab_ref.shape[0]
    doh = (d_ref[...] == lax.broadcasted_iota(jnp.int32, (bt, ncat), 1)).astype(f32)
    hoh = (h_ref[...] == lax.broadcasted_iota(jnp.int32, (bt, ncat), 1)).astype(f32)
    demb = jnp.dot(doh, dtab_ref[...], preferred_element_type=f32)
    hemb = jnp.dot(hoh, htab_ref[...], preferred_element_type=f32)
    acc = jnp.dot(u_ref[...], w1u_ref[...], preferred_element_type=f32)
    acc = acc + jnp.dot(demb, w1d_ref[...], preferred_element_type=f32)
    acc = acc + jnp.dot(hemb, w1h_ref[...], preferred_element_type=f32)
    acc = acc + m_ref[...] * w1m_ref[...]
    h1 = jnp.maximum(acc + b1_ref[...], 0.0)
    h2 = jnp.maximum(
        jnp.dot(h1, w2_ref[...], preferred_element_type=f32) + b2_ref[...], 0.0)
    o_ref[...] = jnp.dot(h2, w3_ref[...], preferred_element_type=f32) + b3_ref[...]


def _mlp_call(uemb, days2, hours2, md2, dtab, htab, w1u, w1d, w1h, w1m, b1r,
              w2, b2r, w3, b3r):
    B = uemb.shape[0]
    n_out = w3.shape[1]
    bt = _BT
    grid = (B // bt,)

    def row_block(cols):
        return pl.BlockSpec((bt, cols), lambda i: (i, 0))

    def full(a):
        return pl.BlockSpec(a.shape, lambda i: (0,) * a.ndim)

    return pl.pallas_call(
        _mlp_body,
        grid=grid,
        in_specs=[
            row_block(uemb.shape[1]),
            row_block(1), row_block(1), row_block(1),
            full(dtab), full(htab), full(w1u), full(w1d), full(w1h),
            full(w1m), full(b1r), full(w2), full(b2r), full(w3), full(b3r),
        ],
        out_specs=row_block(n_out),
        out_shape=jax.ShapeDtypeStruct((B, n_out), jnp.float32),
    )(uemb, days2, hours2, md2, dtab, htab, w1u, w1d, w1h, w1m, b1r, w2, b2r,
      w3, b3r)


def kernel(user_ids, hours, days, move_distance, user_table, day_table,
           hour_table, W1, b1, W2, b2, W3, b3):
    B = user_ids.shape[0]
    D = user_table.shape[1]
    f32 = jnp.float32

    uemb = _sc_gather(user_table, user_ids.astype(jnp.int32))

    ncat = 32
    dtab = jnp.zeros((ncat, D), f32).at[: day_table.shape[0]].set(day_table)
    htab = jnp.zeros((ncat, D), f32).at[: hour_table.shape[0]].set(hour_table)

    w1u = W1[:, :D].T
    w1d = W1[:, D:2 * D].T
    w1h = W1[:, 2 * D:3 * D].T
    w1m = W1[:, 3 * D][None, :]
    b1r = b1[None, :]
    w2 = W2.T
    b2r = b2[None, :]
    n_out = 8
    w3 = jnp.zeros((W2.shape[0], n_out), f32).at[:, : W3.shape[0]].set(W3.T)
    b3r = jnp.zeros((1, n_out), f32).at[0, : W3.shape[0]].set(b3)

    days2 = days.astype(jnp.int32)[:, None]
    hours2 = hours.astype(jnp.int32)[:, None]
    md2 = move_distance[:, None]

    out = _mlp_call(uemb, days2, hours2, md2, dtab, htab, w1u, w1d, w1h, w1m,
                    b1r, w2, b2r, w3, b3r)
    return out[:, : W3.shape[0]]
